# baseline (speedup 1.0000x reference)
import functools

import numpy as np
import jax
import jax.numpy as jnp
from jax import lax
from jax.experimental import pallas as pl
from jax.experimental.pallas import tpu as pltpu
from jax.experimental.pallas import tpu_sc as plsc

N_NODES = 10000
N_PAD = 10240
D = 128

NC = 2
NS = 16
NW = NC * NS
L = 16

CHUNK = 128
G = 8

_GATHER_DNUMS = jax.lax.GatherDimensionNumbers(
    offset_dims=(), collapsed_slice_dims=(0,), start_index_map=(0,))
ROWS_PER_TILE = N_PAD // NS
ZROWS = 128


def _mm_kernel(x_ref, w_ref, o_ref):
    o_ref[...] = jnp.dot(x_ref[...], w_ref[...],
                         preferred_element_type=jnp.float32)


def _mid_kernel(p_ref, w_ref, o_ref):
    h = jnp.maximum(p_ref[0] + p_ref[1], 0.0)
    o_ref[...] = jnp.dot(h, w_ref[...], preferred_element_type=jnp.float32)


def _relu_sum_kernel(p_ref, o_ref):
    o_ref[...] = jnp.maximum(p_ref[0] + p_ref[1], 0.0)


_NBLK = 25
_BLK = N_NODES // _NBLK


def _tc_matmul(x, w):
    return pl.pallas_call(
        _mm_kernel,
        grid=(_NBLK,),
        in_specs=[pl.BlockSpec((_BLK, D), lambda i: (i, 0)),
                  pl.BlockSpec((D, D), lambda i: (0, 0))],
        out_specs=pl.BlockSpec((_BLK, D), lambda i: (i, 0)),
        out_shape=jax.ShapeDtypeStruct((N_NODES, D), jnp.float32),
    )(x, w)


def _tc_mid(p, w):
    return pl.pallas_call(
        _mid_kernel,
        grid=(_NBLK,),
        in_specs=[pl.BlockSpec((NC, _BLK, D), lambda i: (0, i, 0)),
                  pl.BlockSpec((D, D), lambda i: (0, 0))],
        out_specs=pl.BlockSpec((_BLK, D), lambda i: (i, 0)),
        out_shape=jax.ShapeDtypeStruct((N_NODES, D), jnp.float32),
    )(p, w)


def _tc_relu_sum(p):
    return pl.pallas_call(
        _relu_sum_kernel,
        grid=(_NBLK,),
        in_specs=[pl.BlockSpec((NC, _BLK, D), lambda i: (0, i, 0))],
        out_specs=pl.BlockSpec((_BLK, D), lambda i: (i, 0)),
        out_shape=jax.ShapeDtypeStruct((N_NODES, D), jnp.float32),
    )(p)


def _make_sc_spmm(ng):
    mesh = plsc.VectorSubcoreMesh(core_axis_name="c", subcore_axis_name="s")

    @functools.partial(
        pl.kernel,
        out_type=jax.ShapeDtypeStruct((NC, N_PAD, D), jnp.float32),
        mesh=mesh,
        scratch_types=[
            pltpu.VMEM((2, G, CHUNK), jnp.int32),
            pltpu.VMEM((2, G, CHUNK), jnp.int32),
            pltpu.VMEM((2, G, CHUNK), jnp.float32),
            pltpu.VMEM((2, CHUNK, D), jnp.float32),
            pltpu.VMEM_SHARED((N_PAD, D), jnp.float32),
            pltpu.SemaphoreType.DMA,
        ],
    )
    def spmm(h_hbm, src_hbm, dst_hbm, w_hbm, out_hbm,
             sg, dg, wg, rows, acc_sh, sem):
        cid = lax.axis_index("c")
        sid = lax.axis_index("s")
        wid = cid * NS + sid
        nch = ng * G

        zvec = jnp.zeros((L,), jnp.float32)

        def zero_body(r, carry):
            for j in range(D // L):
                rows[0, r, pl.ds(j * L, L)] = zvec
            return carry
        lax.fori_loop(0, ZROWS, zero_body, 0)
        row0 = sid * ROWS_PER_TILE
        for z in range(ROWS_PER_TILE // ZROWS):
            pltpu.sync_copy(rows.at[0],
                            acc_sh.at[pl.ds(row0 + z * ZROWS, ZROWS)])
        plsc.subcore_barrier()

        zlanes = lax.iota(jnp.int32, L) * 0

        def stage_group(gidx, q):
            pltpu.sync_copy(src_hbm.at[wid, gidx], sg.at[q])
            pltpu.sync_copy(dst_hbm.at[wid, gidx], dg.at[q])
            pltpu.sync_copy(w_hbm.at[wid, gidx], wg.at[q])

        stage_group(0, 0)
        pltpu.async_copy(h_hbm.at[sg.at[0, 0]], rows.at[0], sem)

        def chunk_body(c, carry):
            p = c & 1
            g = c >> 3
            q = g & 1
            slot = c & 7

            @pl.when(jnp.logical_and(slot == 0, g + 1 < ng))
            def _():
                stage_group(g + 1, 1 - q)

            pltpu.make_async_copy(
                h_hbm.at[sg.at[q, slot]], rows.at[p], sem).wait()

            @pl.when(c + 1 < nch)
            def _():
                c1 = c + 1
                pltpu.async_copy(
                    h_hbm.at[sg.at[(c1 >> 3) & 1, c1 & 7]],
                    rows.at[1 - p], sem)

            def grp_body(g16, carry2):
                wgrp = wg[q, slot, pl.ds(g16 * L, L)]
                for e16 in range(L):
                    idx = (zlanes + e16).reshape(L, 1)
                    wvec = lax.gather(
                        wgrp, idx,
                        _GATHER_DNUMS, slice_sizes=(1,),
                        mode=lax.GatherScatterMode.PROMISE_IN_BOUNDS)
                    e = g16 * L + e16
                    for j in range(D // L):
                        sl = pl.ds(j * L, L)
                        rows[p, e, sl] = rows[p, e, sl] * wvec
                return carry2
            lax.fori_loop(0, CHUNK // L, grp_body, 0)

            pltpu.sync_copy(rows.at[p], acc_sh.at[dg.at[q, slot]], add=True)
            return carry
        lax.fori_loop(0, nch, chunk_body, 0)
        plsc.subcore_barrier()

        pltpu.sync_copy(acc_sh.at[pl.ds(row0, ROWS_PER_TILE)],
                        out_hbm.at[cid, pl.ds(row0, ROWS_PER_TILE)])

    return spmm


def kernel(x, edge_index, edge_weight, nodes_mask, W0, W1):
    del nodes_mask
    n_edges = edge_index.shape[1]
    gsz = G * CHUNK
    per_tile = -(-n_edges // (NW * gsz)) * gsz
    ng = per_tile // gsz
    ep = NW * per_tile
    pad = ep - n_edges

    src = edge_index[0].astype(jnp.int32)
    dst = edge_index[1].astype(jnp.int32)
    w = edge_weight.astype(jnp.float32)
    if pad:
        zpad = jnp.zeros((pad,), jnp.int32)
        src = jnp.concatenate([src, zpad])
        dst = jnp.concatenate([dst, zpad])
        w = jnp.concatenate([w, jnp.zeros((pad,), jnp.float32)])
    src = src.reshape(NW, ng, G, CHUNK)
    dst = dst.reshape(NW, ng, G, CHUNK)
    w = w.reshape(NW, ng, G, CHUNK)

    spmm = _make_sc_spmm(ng)

    h0 = _tc_matmul(x, W0)
    p = spmm(h0, src, dst, w)
    h1 = _tc_mid(p, W1)
    p2 = spmm(h1, src, dst, w)
    return _tc_relu_sum(p2)


# speedup vs baseline: 1.1600x; 1.1600x over previous
import functools

import numpy as np
import jax
import jax.numpy as jnp
from jax import lax
from jax.experimental import pallas as pl
from jax.experimental.pallas import tpu as pltpu
from jax.experimental.pallas import tpu_sc as plsc

N_NODES = 10000
N_PAD = 10240
D = 128

NC = 2
NS = 16
NW = NC * NS
L = 16

CHUNK = 128
G = 8

_GATHER_DNUMS = jax.lax.GatherDimensionNumbers(
    offset_dims=(), collapsed_slice_dims=(0,), start_index_map=(0,))
ROWS_PER_TILE = N_PAD // NS
ZROWS = 128


def _mm_kernel(x_ref, w_ref, o_ref):
    o_ref[...] = jnp.dot(x_ref[...], w_ref[...],
                         preferred_element_type=jnp.float32)


def _mid_kernel(p_ref, w_ref, o_ref):
    h = jnp.maximum(p_ref[0] + p_ref[1], 0.0)
    o_ref[...] = jnp.dot(h, w_ref[...], preferred_element_type=jnp.float32)


def _relu_sum_kernel(p_ref, o_ref):
    o_ref[...] = jnp.maximum(p_ref[0] + p_ref[1], 0.0)


_NBLK = 25
_BLK = N_NODES // _NBLK


def _tc_matmul(x, w):
    return pl.pallas_call(
        _mm_kernel,
        grid=(_NBLK,),
        in_specs=[pl.BlockSpec((_BLK, D), lambda i: (i, 0)),
                  pl.BlockSpec((D, D), lambda i: (0, 0))],
        out_specs=pl.BlockSpec((_BLK, D), lambda i: (i, 0)),
        out_shape=jax.ShapeDtypeStruct((N_NODES, D), jnp.float32),
    )(x, w)


def _tc_mid(p, w):
    return pl.pallas_call(
        _mid_kernel,
        grid=(_NBLK,),
        in_specs=[pl.BlockSpec((NC, _BLK, D), lambda i: (0, i, 0)),
                  pl.BlockSpec((D, D), lambda i: (0, 0))],
        out_specs=pl.BlockSpec((_BLK, D), lambda i: (i, 0)),
        out_shape=jax.ShapeDtypeStruct((N_NODES, D), jnp.float32),
    )(p, w)


def _tc_relu_sum(p):
    return pl.pallas_call(
        _relu_sum_kernel,
        grid=(_NBLK,),
        in_specs=[pl.BlockSpec((NC, _BLK, D), lambda i: (0, i, 0))],
        out_specs=pl.BlockSpec((_BLK, D), lambda i: (i, 0)),
        out_shape=jax.ShapeDtypeStruct((N_NODES, D), jnp.float32),
    )(p)


def _make_sc_spmm(ng):
    mesh = plsc.VectorSubcoreMesh(core_axis_name="c", subcore_axis_name="s")

    @functools.partial(
        pl.kernel,
        out_type=jax.ShapeDtypeStruct((NC, N_PAD, D), jnp.float32),
        mesh=mesh,
        scratch_types=[
            pltpu.VMEM((2, G, CHUNK), jnp.int32),
            pltpu.VMEM((2, G, CHUNK), jnp.int32),
            pltpu.VMEM((2, G, CHUNK), jnp.float32),
            pltpu.VMEM((2, CHUNK, D), jnp.float32),
            pltpu.VMEM_SHARED((N_PAD, D), jnp.float32),
            pltpu.SemaphoreType.DMA,
            pltpu.SemaphoreType.DMA,
        ],
    )
    def spmm(h_hbm, src_hbm, dst_hbm, w_hbm, out_hbm,
             sg, dg, wg, rows, acc_sh, sem, sem_i):
        cid = lax.axis_index("c")
        sid = lax.axis_index("s")
        wid = cid * NS + sid
        nch = ng * G

        def stage_group_start(gidx, q):
            pltpu.async_copy(src_hbm.at[wid, gidx], sg.at[q], sem_i)
            pltpu.async_copy(dst_hbm.at[wid, gidx], dg.at[q], sem_i)
            pltpu.async_copy(w_hbm.at[wid, gidx], wg.at[q], sem_i)

        def stage_group_wait(gidx, q):
            pltpu.make_async_copy(src_hbm.at[wid, gidx], sg.at[q],
                                  sem_i).wait()
            pltpu.make_async_copy(dst_hbm.at[wid, gidx], dg.at[q],
                                  sem_i).wait()
            pltpu.make_async_copy(w_hbm.at[wid, gidx], wg.at[q],
                                  sem_i).wait()

        stage_group_start(0, 0)

        zvec = jnp.zeros((L,), jnp.float32)

        def zero_body(r, carry):
            for j in range(D // L):
                rows[0, r, pl.ds(j * L, L)] = zvec
            return carry
        lax.fori_loop(0, CHUNK, zero_body, 0)
        row0 = sid * ROWS_PER_TILE
        for z in range(ROWS_PER_TILE // CHUNK):
            pltpu.sync_copy(rows.at[0],
                            acc_sh.at[pl.ds(row0 + z * CHUNK, CHUNK)])
        plsc.subcore_barrier()

        zlanes = lax.iota(jnp.int32, L) * 0

        stage_group_wait(0, 0)
        pltpu.async_copy(h_hbm.at[sg.at[0, 0]], rows.at[0], sem)

        def pair_body(k, carry):
            for p in (0, 1):
                c = 2 * k + p
                g = c >> 3
                q = g & 1
                slot = c & 7

                if p == 0:
                    @pl.when(jnp.logical_and(slot == 0, g + 1 < ng))
                    def _():
                        stage_group_start(g + 1, 1 - q)

                pltpu.make_async_copy(
                    h_hbm.at[sg.at[q, slot]], rows.at[p], sem).wait()

                @pl.when(c + 1 < nch)
                def _():
                    c1 = c + 1
                    q1 = (c1 >> 3) & 1

                    @pl.when((c1 & 7) == 0)
                    def _():
                        stage_group_wait(c1 >> 3, q1)
                    pltpu.async_copy(
                        h_hbm.at[sg.at[q1, c1 & 7]], rows.at[1 - p], sem)

                def grp_body(g16, carry2):
                    wgrp = wg[q, slot, pl.ds(g16 * L, L)]
                    for e16 in range(L):
                        idx = (zlanes + e16).reshape(L, 1)
                        wvec = lax.gather(
                            wgrp, idx,
                            _GATHER_DNUMS, slice_sizes=(1,),
                            mode=lax.GatherScatterMode.PROMISE_IN_BOUNDS)
                        e = g16 * L + e16
                        for j in range(D // L):
                            sl = pl.ds(j * L, L)
                            rows[p, e, sl] = rows[p, e, sl] * wvec
                    return carry2
                lax.fori_loop(0, CHUNK // L, grp_body, 0)

                pltpu.sync_copy(rows.at[p], acc_sh.at[dg.at[q, slot]],
                                add=True)
            return carry
        lax.fori_loop(0, nch // 2, pair_body, 0)
        plsc.subcore_barrier()

        pltpu.sync_copy(acc_sh.at[pl.ds(row0, ROWS_PER_TILE)],
                        out_hbm.at[cid, pl.ds(row0, ROWS_PER_TILE)])

    return spmm


def kernel(x, edge_index, edge_weight, nodes_mask, W0, W1):
    del nodes_mask
    n_edges = edge_index.shape[1]
    gsz = G * CHUNK
    per_tile = -(-n_edges // (NW * gsz)) * gsz
    ng = per_tile // gsz
    ep = NW * per_tile
    pad = ep - n_edges

    src = edge_index[0].astype(jnp.int32)
    dst = edge_index[1].astype(jnp.int32)
    w = edge_weight.astype(jnp.float32)
    if pad:
        zpad = jnp.zeros((pad,), jnp.int32)
        src = jnp.concatenate([src, zpad])
        dst = jnp.concatenate([dst, zpad])
        w = jnp.concatenate([w, jnp.zeros((pad,), jnp.float32)])
    src = src.reshape(NW, ng, G, CHUNK)
    dst = dst.reshape(NW, ng, G, CHUNK)
    w = w.reshape(NW, ng, G, CHUNK)

    spmm = _make_sc_spmm(ng)

    h0 = _tc_matmul(x, W0)
    p = spmm(h0, src, dst, w)
    h1 = _tc_mid(p, W1)
    p2 = spmm(h1, src, dst, w)
    return _tc_relu_sum(p2)
